# baseline (speedup 1.0000x reference)
import functools

import jax
import jax.numpy as jnp
from jax import lax
from jax.experimental import pallas as pl
from jax.experimental.pallas import tpu as pltpu
from jax.experimental.pallas import tpu_sc as plsc

D = 768
NV = D // 16
EPS = 1e-12
_NC, _NS = 2, 16
_NW = _NC * _NS
_CH = 32

_INV_D = 1.0 / D
_MAGIC = 0x5F3759DF


_DNUMS = lax.GatherDimensionNumbers(
    offset_dims=(), collapsed_slice_dims=(0,), start_index_map=(0,))


def _permute(x, idx):
    return lax.gather(x, idx.reshape(16, 1), _DNUMS, (1,),
                      mode=lax.GatherScatterMode.PROMISE_IN_BOUNDS)


def _lane_sum(x):
    for k in (8, 4, 2, 1):
        x = x + _permute(x, lax.iota(jnp.int32, 16) ^ k)
    return x


def _rsqrt16(a):
    i = lax.bitcast_convert_type(a, jnp.int32)
    y = lax.bitcast_convert_type(_MAGIC - (i >> 1), jnp.float32)
    for _ in range(3):
        y = y * (1.5 - 0.5 * a * y * y)
    return y


def _sc_embed_ln(word_table, type_table, gamma, idx3, tti2):
    nw, n_ch, ch = idx3.shape
    b_total = nw * n_ch * ch
    n_half = n_ch // 2
    mesh = plsc.VectorSubcoreMesh(core_axis_name="c", subcore_axis_name="s")

    @functools.partial(
        pl.kernel,
        mesh=mesh,
        out_type=jax.ShapeDtypeStruct((b_total, D), jnp.float32),
        scratch_types=[
            pltpu.VMEM((n_ch, ch), jnp.int32),
            pltpu.VMEM((n_ch * ch,), jnp.int32),
            pltpu.VMEM((2, D), jnp.float32),
            pltpu.VMEM((D,), jnp.float32),
            pltpu.VMEM((D,), jnp.float32),
            pltpu.VMEM((ch, D), jnp.float32),
            pltpu.VMEM((ch, D), jnp.float32),
            pltpu.VMEM((ch, D), jnp.float32),
            pltpu.VMEM((ch, D), jnp.float32),
            pltpu.SemaphoreType.DMA,
            pltpu.SemaphoreType.DMA,
            pltpu.SemaphoreType.DMA,
            pltpu.SemaphoreType.DMA,
        ],
    )
    def k(table_hbm, ttab_hbm, gam_hbm, idx_hbm, tti_hbm, out_hbm,
          idx_v, tti_v, ttab_v, dtab_v, gam_v, gb0, gb1, ob0, ob1,
          gs0, gs1, os0, os1):
        wid = lax.axis_index("s") * _NC + lax.axis_index("c")
        base = wid * (n_ch * ch)
        pltpu.sync_copy(idx_hbm.at[wid], idx_v)
        pltpu.sync_copy(tti_hbm.at[wid], tti_v)
        pltpu.sync_copy(ttab_hbm, ttab_v)
        pltpu.sync_copy(gam_hbm, gam_v)
        for j in range(NV):
            sl = pl.ds(16 * j, 16)
            dtab_v[sl] = ttab_v[1, sl] - ttab_v[0, sl]
        gbufs = (gb0, gb1)
        obufs = (ob0, ob1)
        gsems = (gs0, gs1)
        osems = (os0, os1)

        pltpu.async_copy(table_hbm.at[idx_v.at[0]], gb0, gs0)
        pltpu.async_copy(table_hbm.at[idx_v.at[1]], gb1, gs1)

        def chunk(i, b, c):
            gb, ob = gbufs[b], obufs[b]
            pltpu.make_async_copy(table_hbm.at[idx_v.at[c]], gb, gsems[b]).wait()

            @pl.when(i >= 1)
            def _():
                pltpu.make_async_copy(
                    ob, out_hbm.at[pl.ds(base + (c - 2) * ch, ch)],
                    osems[b]).wait()

            def row(r, carry):
                pos = c * ch + r
                grp = tti_v[pl.ds(pos & ~15, 16)]
                ttf = _permute(grp, lax.broadcast(pos & 15, (16,))
                               ).astype(jnp.float32)
                sum_v = jnp.zeros((16,), jnp.float32)
                sq_v = jnp.zeros((16,), jnp.float32)
                xs = []
                for j in range(NV):
                    sl = pl.ds(16 * j, 16)
                    x = gb[r, sl] + (ttab_v[0, sl] + ttf * dtab_v[sl])
                    xs.append(x)
                    sum_v = sum_v + x
                    sq_v = sq_v + x * x
                mean = _lane_sum(sum_v) * _INV_D
                msq = _lane_sum(sq_v) * _INV_D
                rs = _rsqrt16(msq - mean * mean + EPS)
                bia = mean * rs
                for j in range(NV):
                    sl = pl.ds(16 * j, 16)
                    ob[r, sl] = (xs[j] * rs - bia) * gam_v[sl]
                return carry

            lax.fori_loop(0, ch, row, 0)

            pltpu.async_copy(ob, out_hbm.at[pl.ds(base + c * ch, ch)],
                             osems[b])

            @pl.when(i < n_half - 1)
            def _():
                pltpu.async_copy(table_hbm.at[idx_v.at[c + 2]], gb, gsems[b])

        def body(i, carry):
            chunk(i, 0, 2 * i)
            chunk(i, 1, 2 * i + 1)
            return carry

        lax.fori_loop(0, n_half, body, 0)

        pltpu.make_async_copy(
            ob0, out_hbm.at[pl.ds(base + (n_ch - 2) * ch, ch)], os0).wait()
        pltpu.make_async_copy(
            ob1, out_hbm.at[pl.ds(base + (n_ch - 1) * ch, ch)], os1).wait()

    return k(word_table, type_table, gamma, idx3, tti2)


def kernel(input_ids, token_type_ids, word_table, type_table, gamma):
    batch, seq = input_ids.shape
    b_total = batch * seq
    n_ch = b_total // (_NW * _CH)
    idx3 = input_ids.reshape(_NW, n_ch, _CH)
    tti2 = token_type_ids.reshape(_NW, n_ch * _CH)
    out = _sc_embed_ln(word_table, type_table, gamma, idx3, tti2)
    return out.reshape(batch, seq, D)


# speedup vs baseline: 2.1487x; 2.1487x over previous
import functools

import jax
import jax.numpy as jnp
from jax import lax
from jax.experimental import pallas as pl
from jax.experimental.pallas import tpu as pltpu
from jax.experimental.pallas import tpu_sc as plsc

D = 768
EPS = 1e-12
_NC, _NS = 2, 16
_NW = _NC * _NS
_CH = 64


def _sc_gather(word_table, idx3):
    nw, n_ch, ch = idx3.shape
    b_total = nw * n_ch * ch
    mesh = plsc.VectorSubcoreMesh(core_axis_name="c", subcore_axis_name="s")

    @functools.partial(
        pl.kernel,
        mesh=mesh,
        out_type=jax.ShapeDtypeStruct((b_total, D), jnp.float32),
        scratch_types=[
            pltpu.VMEM((n_ch, ch), jnp.int32),
            pltpu.VMEM((ch, D), jnp.float32),
            pltpu.VMEM((ch, D), jnp.float32),
            pltpu.SemaphoreType.DMA,
            pltpu.SemaphoreType.DMA,
            pltpu.SemaphoreType.DMA,
            pltpu.SemaphoreType.DMA,
        ],
    )
    def k(table_hbm, idx_hbm, out_hbm, idx_v, buf0, buf1, g0, g1, o0, o1):
        wid = lax.axis_index("s") * _NC + lax.axis_index("c")
        base = wid * (n_ch * ch)
        pltpu.sync_copy(idx_hbm.at[wid], idx_v)
        bufs = (buf0, buf1)
        gsems = (g0, g1)
        osems = (o0, o1)
        gh = [None] * n_ch
        oh = [None] * n_ch
        gh[0] = pltpu.async_copy(table_hbm.at[idx_v.at[0]], bufs[0], gsems[0])
        if n_ch > 1:
            gh[1] = pltpu.async_copy(table_hbm.at[idx_v.at[1]], bufs[1], gsems[1])
        for c in range(n_ch):
            b = c % 2
            gh[c].wait()
            oh[c] = pltpu.async_copy(
                bufs[b], out_hbm.at[pl.ds(base + c * ch, ch)], osems[b])
            if c + 2 < n_ch:
                oh[c].wait()
                gh[c + 2] = pltpu.async_copy(
                    table_hbm.at[idx_v.at[c + 2]], bufs[b], gsems[b])
        if n_ch >= 2:
            oh[n_ch - 2].wait()
        oh[n_ch - 1].wait()

    return k(word_table, idx3)


def _ln_body(tt_ref, tab_ref, gamma_ref, x_ref, o_ref):
    x = x_ref[...]
    ttf = tt_ref[0]
    t0 = tab_ref[0, :][None, :]
    dt = tab_ref[1, :][None, :] - t0
    x = x + t0 + ttf * dt
    mean = jnp.mean(x, axis=1, keepdims=True)
    xc = x - mean
    var = jnp.mean(xc * xc, axis=1, keepdims=True)
    o_ref[...] = xc * lax.rsqrt(var + EPS) * gamma_ref[0, :][None, :]


def _tc_layernorm(gathered, token_type_flat, type_table, gamma, tb=1024):
    b_total = gathered.shape[0]
    nb = b_total // tb
    tt3 = token_type_flat.reshape(nb, tb, 1).astype(jnp.float32)
    gamma2 = gamma.reshape(1, D)
    return pl.pallas_call(
        _ln_body,
        grid=(nb,),
        in_specs=[
            pl.BlockSpec((1, tb, 1), lambda i: (i, 0, 0)),
            pl.BlockSpec((2, D), lambda i: (0, 0)),
            pl.BlockSpec((1, D), lambda i: (0, 0)),
            pl.BlockSpec((tb, D), lambda i: (i, 0)),
        ],
        out_specs=pl.BlockSpec((tb, D), lambda i: (i, 0)),
        out_shape=jax.ShapeDtypeStruct((b_total, D), jnp.float32),
    )(tt3, type_table, gamma2, gathered)


def kernel(input_ids, token_type_ids, word_table, type_table, gamma):
    batch, seq = input_ids.shape
    b_total = batch * seq
    n_ch = b_total // (_NW * _CH)
    idx3 = input_ids.reshape(_NW, n_ch, _CH)
    gathered = _sc_gather(word_table, idx3)
    out = _tc_layernorm(gathered, token_type_ids.reshape(-1), type_table, gamma)
    return out.reshape(batch, seq, D)


# speedup vs baseline: 2.2145x; 1.0306x over previous
import functools

import jax
import jax.numpy as jnp
from jax import lax
from jax.experimental import pallas as pl
from jax.experimental.pallas import tpu as pltpu
from jax.experimental.pallas import tpu_sc as plsc

D = 768
EPS = 1e-12
_NC, _NS = 2, 16
_NW = _NC * _NS
_CH = 64


def _sc_gather(word_table, idx3):
    nw, n_ch, ch = idx3.shape
    b_total = nw * n_ch * ch
    mesh = plsc.VectorSubcoreMesh(core_axis_name="c", subcore_axis_name="s")

    @functools.partial(
        pl.kernel,
        mesh=mesh,
        out_type=jax.ShapeDtypeStruct((b_total, D), jnp.float32),
        scratch_types=[
            pltpu.VMEM((n_ch, ch), jnp.int32),
            pltpu.VMEM((ch, D), jnp.float32),
            pltpu.VMEM((ch, D), jnp.float32),
            pltpu.SemaphoreType.DMA,
            pltpu.SemaphoreType.DMA,
            pltpu.SemaphoreType.DMA,
            pltpu.SemaphoreType.DMA,
        ],
    )
    def k(table_hbm, idx_hbm, out_hbm, idx_v, buf0, buf1, g0, g1, o0, o1):
        wid = lax.axis_index("s") * _NC + lax.axis_index("c")
        base = wid * (n_ch * ch)
        pltpu.sync_copy(idx_hbm.at[wid], idx_v)
        bufs = (buf0, buf1)
        gsems = (g0, g1)
        osems = (o0, o1)
        gh = [None] * n_ch
        oh = [None] * n_ch
        gh[0] = pltpu.async_copy(table_hbm.at[idx_v.at[0]], bufs[0], gsems[0])
        if n_ch > 1:
            gh[1] = pltpu.async_copy(table_hbm.at[idx_v.at[1]], bufs[1], gsems[1])
        for c in range(n_ch):
            b = c % 2
            gh[c].wait()
            oh[c] = pltpu.async_copy(
                bufs[b], out_hbm.at[pl.ds(base + c * ch, ch)], osems[b])
            if c + 2 < n_ch:
                oh[c].wait()
                gh[c + 2] = pltpu.async_copy(
                    table_hbm.at[idx_v.at[c + 2]], bufs[b], gsems[b])
        if n_ch >= 2:
            oh[n_ch - 2].wait()
        oh[n_ch - 1].wait()

    return k(word_table, idx3)


def _ln_body(tt_ref, tab_ref, gamma_ref, x_ref, o_ref):
    x = x_ref[...]
    ttf = tt_ref[0]
    t0 = tab_ref[0, :][None, :]
    dt = tab_ref[1, :][None, :] - t0
    x = x + t0 + ttf * dt
    mean = jnp.mean(x, axis=1, keepdims=True)
    xc = x - mean
    var = jnp.mean(xc * xc, axis=1, keepdims=True)
    o_ref[...] = xc * lax.rsqrt(var + EPS) * gamma_ref[0, :][None, :]


def _tc_layernorm(gathered, token_type_flat, type_table, gamma, tb=2048):
    b_total = gathered.shape[0]
    nb = b_total // tb
    tt3 = token_type_flat.reshape(nb, tb, 1).astype(jnp.float32)
    gamma2 = gamma.reshape(1, D)
    return pl.pallas_call(
        _ln_body,
        grid=(nb,),
        in_specs=[
            pl.BlockSpec((1, tb, 1), lambda i: (i, 0, 0)),
            pl.BlockSpec((2, D), lambda i: (0, 0)),
            pl.BlockSpec((1, D), lambda i: (0, 0)),
            pl.BlockSpec((tb, D), lambda i: (i, 0)),
        ],
        out_specs=pl.BlockSpec((tb, D), lambda i: (i, 0)),
        out_shape=jax.ShapeDtypeStruct((b_total, D), jnp.float32),
    )(tt3, type_table, gamma2, gathered)


def kernel(input_ids, token_type_ids, word_table, type_table, gamma):
    batch, seq = input_ids.shape
    b_total = batch * seq
    n_ch = b_total // (_NW * _CH)
    idx3 = input_ids.reshape(_NW, n_ch, _CH)
    gathered = _sc_gather(word_table, idx3)
    out = _tc_layernorm(gathered, token_type_ids.reshape(-1), type_table, gamma)
    return out.reshape(batch, seq, D)
